# speedup vs baseline: 1.6695x; 1.6695x over previous
import jax
import jax.numpy as jnp
from jax.experimental import pallas as pl

_NUM_EXPERTS = 16
_BT = 512


def _router_block(x_ref, w1_ref, b1_ref, w2_ref, b2_ref,
                  ori_ref, router_ref, idx_ref):
    h = jnp.maximum(
        jnp.dot(x_ref[...], w1_ref[...],
                preferred_element_type=jnp.float32) + b1_ref[...],
        0.0)
    logits = jnp.dot(h, w2_ref[...],
                     preferred_element_type=jnp.float32) + b2_ref[...]

    iota = jax.lax.broadcasted_iota(jnp.int32, logits.shape, 1)
    m1 = jnp.max(logits, axis=1, keepdims=True)
    idx1 = jnp.min(jnp.where(logits == m1, iota, _NUM_EXPERTS),
                   axis=1, keepdims=True)
    neg_inf = jnp.float32(-jnp.inf)
    masked = jnp.where(iota == idx1, neg_inf, logits)
    m2 = jnp.max(masked, axis=1, keepdims=True)
    idx2 = jnp.min(jnp.where(masked == m2, iota, _NUM_EXPERTS),
                   axis=1, keepdims=True)

    t = (logits - m1) * 100.0
    e = jnp.exp(t)
    ori_ref[...] = e / jnp.sum(e, axis=1, keepdims=True)

    p2 = jnp.exp(m2 - m1)
    den = 1.0 + p2
    router_ref[...] = jnp.where(
        iota == idx1, 1.0 / den,
        jnp.where(iota == idx2, p2 / den, 0.0))

    idx_ref[...] = jnp.concatenate([idx1, idx2], axis=1)


def kernel(x, W1, b1, W2, b2):
    tokens, input_dim = x.shape
    hidden = W1.shape[1]
    grid = (tokens // _BT,)
    b1r = b1.reshape(1, hidden)
    b2r = b2.reshape(1, _NUM_EXPERTS)
    out_shapes = (
        jax.ShapeDtypeStruct((tokens, _NUM_EXPERTS), jnp.float32),
        jax.ShapeDtypeStruct((tokens, _NUM_EXPERTS), jnp.float32),
        jax.ShapeDtypeStruct((tokens, 2), jnp.int32),
    )
    ori, router, idx = pl.pallas_call(
        _router_block,
        grid=grid,
        in_specs=[
            pl.BlockSpec((_BT, input_dim), lambda i: (i, 0)),
            pl.BlockSpec((input_dim, hidden), lambda i: (0, 0)),
            pl.BlockSpec((1, hidden), lambda i: (0, 0)),
            pl.BlockSpec((hidden, _NUM_EXPERTS), lambda i: (0, 0)),
            pl.BlockSpec((1, _NUM_EXPERTS), lambda i: (0, 0)),
        ],
        out_specs=(
            pl.BlockSpec((_BT, _NUM_EXPERTS), lambda i: (i, 0)),
            pl.BlockSpec((_BT, _NUM_EXPERTS), lambda i: (i, 0)),
            pl.BlockSpec((_BT, 2), lambda i: (i, 0)),
        ),
        out_shape=out_shapes,
    )(x, W1, b1r, W2, b2r)
    return (ori, router, idx)


# speedup vs baseline: 1.7780x; 1.0650x over previous
import jax
import jax.numpy as jnp
from jax.experimental import pallas as pl

_NUM_EXPERTS = 16
_BT = 1024


def _router_block(x_ref, w1_ref, b1_ref, w2_ref, b2_ref,
                  ori_ref, router_ref, idx_ref):
    h = jnp.maximum(
        jnp.dot(x_ref[...], w1_ref[...],
                preferred_element_type=jnp.float32) + b1_ref[...],
        0.0)
    logits = jnp.dot(h, w2_ref[...],
                          preferred_element_type=jnp.float32) + b2_ref[...]

    iota = jax.lax.broadcasted_iota(jnp.int32, logits.shape, 1)
    m1 = jnp.max(logits, axis=1, keepdims=True)
    idx1 = jnp.min(jnp.where(logits == m1, iota, _NUM_EXPERTS),
                   axis=1, keepdims=True)
    neg_inf = jnp.float32(-jnp.inf)
    masked = jnp.where(iota == idx1, neg_inf, logits)
    m2 = jnp.max(masked, axis=1, keepdims=True)
    idx2 = jnp.min(jnp.where(masked == m2, iota, _NUM_EXPERTS),
                   axis=1, keepdims=True)

    t = (logits - m1) * 100.0
    e = jnp.exp(t)
    ori_ref[...] = e / jnp.sum(e, axis=1, keepdims=True)

    p2 = jnp.exp(m2 - m1)
    den = 1.0 + p2
    router_ref[...] = jnp.where(
        iota == idx1, 1.0 / den,
        jnp.where(iota == idx2, p2 / den, 0.0))

    idx_ref[...] = jnp.concatenate([idx1, idx2], axis=1)


def kernel(x, W1, b1, W2, b2):
    tokens, input_dim = x.shape
    hidden = W1.shape[1]
    grid = (tokens // _BT,)
    b1r = b1.reshape(1, hidden)
    b2r = b2.reshape(1, _NUM_EXPERTS)
    out_shapes = (
        jax.ShapeDtypeStruct((tokens, _NUM_EXPERTS), jnp.float32),
        jax.ShapeDtypeStruct((tokens, _NUM_EXPERTS), jnp.float32),
        jax.ShapeDtypeStruct((tokens, 2), jnp.int32),
    )
    ori, router, idx = pl.pallas_call(
        _router_block,
        grid=grid,
        in_specs=[
            pl.BlockSpec((_BT, input_dim), lambda i: (i, 0)),
            pl.BlockSpec((input_dim, hidden), lambda i: (0, 0)),
            pl.BlockSpec((1, hidden), lambda i: (0, 0)),
            pl.BlockSpec((hidden, _NUM_EXPERTS), lambda i: (0, 0)),
            pl.BlockSpec((1, _NUM_EXPERTS), lambda i: (0, 0)),
        ],
        out_specs=(
            pl.BlockSpec((_BT, _NUM_EXPERTS), lambda i: (i, 0)),
            pl.BlockSpec((_BT, _NUM_EXPERTS), lambda i: (i, 0)),
            pl.BlockSpec((_BT, 2), lambda i: (i, 0)),
        ),
        out_shape=out_shapes,
    )(x, W1, b1r, W2, b2r)
    return (ori, router, idx)


# speedup vs baseline: 1.7869x; 1.0050x over previous
import jax
import jax.numpy as jnp
from jax.experimental import pallas as pl

_NUM_EXPERTS = 16
_BT = 2048


def _router_block(x_ref, w1_ref, b1_ref, w2_ref, b2_ref,
                  ori_ref, router_ref, idx_ref):
    h = jnp.maximum(
        jnp.dot(x_ref[...], w1_ref[...],
                preferred_element_type=jnp.float32) + b1_ref[...],
        0.0)
    logits = jnp.dot(h, w2_ref[...],
                          preferred_element_type=jnp.float32) + b2_ref[...]

    iota = jax.lax.broadcasted_iota(jnp.int32, logits.shape, 1)
    m1 = jnp.max(logits, axis=1, keepdims=True)
    idx1 = jnp.min(jnp.where(logits == m1, iota, _NUM_EXPERTS),
                   axis=1, keepdims=True)
    neg_inf = jnp.float32(-jnp.inf)
    masked = jnp.where(iota == idx1, neg_inf, logits)
    m2 = jnp.max(masked, axis=1, keepdims=True)
    idx2 = jnp.min(jnp.where(masked == m2, iota, _NUM_EXPERTS),
                   axis=1, keepdims=True)

    t = (logits - m1) * 100.0
    e = jnp.exp(t)
    ori_ref[...] = e / jnp.sum(e, axis=1, keepdims=True)

    p2 = jnp.exp(m2 - m1)
    den = 1.0 + p2
    router_ref[...] = jnp.where(
        iota == idx1, 1.0 / den,
        jnp.where(iota == idx2, p2 / den, 0.0))

    idx_ref[...] = jnp.concatenate([idx1, idx2], axis=1)


def kernel(x, W1, b1, W2, b2):
    tokens, input_dim = x.shape
    hidden = W1.shape[1]
    grid = (tokens // _BT,)
    b1r = b1.reshape(1, hidden)
    b2r = b2.reshape(1, _NUM_EXPERTS)
    out_shapes = (
        jax.ShapeDtypeStruct((tokens, _NUM_EXPERTS), jnp.float32),
        jax.ShapeDtypeStruct((tokens, _NUM_EXPERTS), jnp.float32),
        jax.ShapeDtypeStruct((tokens, 2), jnp.int32),
    )
    ori, router, idx = pl.pallas_call(
        _router_block,
        grid=grid,
        in_specs=[
            pl.BlockSpec((_BT, input_dim), lambda i: (i, 0)),
            pl.BlockSpec((input_dim, hidden), lambda i: (0, 0)),
            pl.BlockSpec((1, hidden), lambda i: (0, 0)),
            pl.BlockSpec((hidden, _NUM_EXPERTS), lambda i: (0, 0)),
            pl.BlockSpec((1, _NUM_EXPERTS), lambda i: (0, 0)),
        ],
        out_specs=(
            pl.BlockSpec((_BT, _NUM_EXPERTS), lambda i: (i, 0)),
            pl.BlockSpec((_BT, _NUM_EXPERTS), lambda i: (i, 0)),
            pl.BlockSpec((_BT, 2), lambda i: (i, 0)),
        ),
        out_shape=out_shapes,
    )(x, W1, b1r, W2, b2r)
    return (ori, router, idx)
